# Initial kernel scaffold; baseline (speedup 1.0000x reference)
import jax
import jax.numpy as jnp
from jax.experimental import pallas as pl


def kernel(indices, embeddings):
    raise NotImplementedError("write your pallas kernel here")


# speedup vs baseline: 2.0391x; 2.0391x over previous
import functools

import jax
import jax.numpy as jnp
from jax import lax
from jax.experimental import pallas as pl
from jax.experimental.pallas import tpu as pltpu
from jax.experimental.pallas import tpu_sc as plsc

_D = 64
_HIST = 50
_LANES = 16
_SEQ_PER_CHUNK = 2
_CHUNK_IDX = _SEQ_PER_CHUNK * _HIST


def _sc_geometry():
    try:
        info = plsc.get_sparse_core_info()
        return info.num_cores, info.num_subcores
    except Exception:
        return 2, 16


@functools.lru_cache(maxsize=None)
def _make_sc_pooler(batch):
    nc, ns = _sc_geometry()
    nw = nc * ns
    seq_per_w = batch // nw
    nchunk = seq_per_w // _SEQ_PER_CHUNK
    mesh = plsc.VectorSubcoreMesh(core_axis_name="c", subcore_axis_name="s")

    @functools.partial(
        pl.kernel,
        mesh=mesh,
        out_type=jax.ShapeDtypeStruct((batch, _D), jnp.float32),
        scratch_types=[
            pltpu.VMEM((nchunk, _CHUNK_IDX), jnp.int32),
            pltpu.VMEM((_CHUNK_IDX, _D), jnp.float32),
            pltpu.VMEM((seq_per_w, _D), jnp.float32),
            pltpu.SemaphoreType.DMA,
        ],
        compiler_params=pltpu.CompilerParams(use_tc_tiling_on_sc=False),
    )
    def pooler(idx_hbm, table_hbm, out_hbm, idx_v, rows_v, out_v, sem):
        wid = lax.axis_index("s") * nc + lax.axis_index("c")
        pltpu.sync_copy(idx_hbm.at[wid], idx_v)

        def chunk_body(i, carry):
            pltpu.async_copy(table_hbm.at[idx_v.at[i]], rows_v, sem).wait()
            for s in range(_SEQ_PER_CHUNK):
                for k in range(_D // _LANES):
                    col = pl.ds(k * _LANES, _LANES)
                    acc = rows_v[s * _HIST, col]
                    for j in range(1, _HIST):
                        acc = acc + rows_v[s * _HIST + j, col]
                    out_v[i * _SEQ_PER_CHUNK + s, col] = acc
            return carry

        lax.fori_loop(0, nchunk, chunk_body, 0)
        pltpu.sync_copy(out_v, out_hbm.at[pl.ds(wid * seq_per_w, seq_per_w)])

    return pooler, nw, nchunk


def _normalize(pooled):
    b = pooled.shape[0]
    blk = 2048

    def body(x_ref, o_ref):
        x = x_ref[...] * (1.0 / _HIST)
        norm = jnp.sqrt(jnp.sum(x * x, axis=1, keepdims=True))
        o_ref[...] = x / jnp.maximum(norm, 1e-12)

    return pl.pallas_call(
        body,
        grid=(b // blk,),
        in_specs=[pl.BlockSpec((blk, _D), lambda i: (i, 0))],
        out_specs=pl.BlockSpec((blk, _D), lambda i: (i, 0)),
        out_shape=jax.ShapeDtypeStruct((b, _D), jnp.float32),
    )(pooled)


def kernel(indices, embeddings):
    b, h = indices.shape
    assert h == _HIST and embeddings.shape[1] == _D
    pooler, nw, nchunk = _make_sc_pooler(b)
    idx3 = indices.astype(jnp.int32).reshape(nw, nchunk, _CHUNK_IDX)
    pooled = pooler(idx3, embeddings.astype(jnp.float32))
    return _normalize(pooled)


# speedup vs baseline: 2.0828x; 1.0214x over previous
import functools

import jax
import jax.numpy as jnp
from jax import lax
from jax.experimental import pallas as pl
from jax.experimental.pallas import tpu as pltpu
from jax.experimental.pallas import tpu_sc as plsc

_D = 64
_HIST = 50
_LANES = 16
_SEQ_PER_CHUNK = 2
_CHUNK_IDX = _SEQ_PER_CHUNK * _HIST
_NBUF = 4


def _sc_geometry():
    try:
        info = plsc.get_sparse_core_info()
        return info.num_cores, info.num_subcores
    except Exception:
        return 2, 16


@functools.lru_cache(maxsize=None)
def _make_sc_pooler(batch):
    nc, ns = _sc_geometry()
    nw = nc * ns
    seq_per_w = batch // nw
    nchunk = seq_per_w // _SEQ_PER_CHUNK
    mesh = plsc.VectorSubcoreMesh(core_axis_name="c", subcore_axis_name="s")

    @functools.partial(
        pl.kernel,
        mesh=mesh,
        out_type=jax.ShapeDtypeStruct((batch, _D), jnp.float32),
        scratch_types=[
            pltpu.VMEM((nchunk, _CHUNK_IDX), jnp.int32),
            *[pltpu.VMEM((_CHUNK_IDX, _D), jnp.float32) for _ in range(_NBUF)],
            pltpu.VMEM((seq_per_w, _D), jnp.float32),
            *[pltpu.SemaphoreType.DMA for _ in range(_NBUF)],
        ],
        compiler_params=pltpu.CompilerParams(use_tc_tiling_on_sc=False),
    )
    def pooler(idx_hbm, table_hbm, out_hbm, idx_v, *rest):
        rows = rest[:_NBUF]
        out_v = rest[_NBUF]
        sems = rest[_NBUF + 1:]
        wid = lax.axis_index("s") * nc + lax.axis_index("c")
        pltpu.sync_copy(idx_hbm.at[wid], idx_v)

        def start(chunk, b):
            pltpu.async_copy(table_hbm.at[idx_v.at[chunk]], rows[b], sems[b])

        def drain(b):
            pltpu.make_async_copy(table_hbm.at[idx_v.at[0]], rows[b], sems[b]).wait()

        def accumulate(chunk, b):
            for s in range(_SEQ_PER_CHUNK):
                for k in range(_D // _LANES):
                    col = pl.ds(k * _LANES, _LANES)
                    acc = rows[b][s * _HIST, col]
                    for j in range(1, _HIST):
                        acc = acc + rows[b][s * _HIST + j, col]
                    out_v[chunk * _SEQ_PER_CHUNK + s, col] = acc

        for b in range(_NBUF):
            start(b, b)

        def group_body(g, carry):
            i = g * _NBUF
            for b in range(_NBUF):
                chunk = i + b
                drain(b)
                accumulate(chunk, b)

                @pl.when(chunk + _NBUF < nchunk)
                def _():
                    start(chunk + _NBUF, b)

            return carry

        lax.fori_loop(0, nchunk // _NBUF, group_body, 0)
        pltpu.sync_copy(out_v, out_hbm.at[pl.ds(wid * seq_per_w, seq_per_w)])

    return pooler, nw, nchunk


def _normalize(pooled):
    b = pooled.shape[0]
    blk = 2048

    def body(x_ref, o_ref):
        x = x_ref[...] * (1.0 / _HIST)
        norm = jnp.sqrt(jnp.sum(x * x, axis=1, keepdims=True))
        o_ref[...] = x / jnp.maximum(norm, 1e-12)

    return pl.pallas_call(
        body,
        grid=(b // blk,),
        in_specs=[pl.BlockSpec((blk, _D), lambda i: (i, 0))],
        out_specs=pl.BlockSpec((blk, _D), lambda i: (i, 0)),
        out_shape=jax.ShapeDtypeStruct((b, _D), jnp.float32),
    )(pooled)


def kernel(indices, embeddings):
    b, h = indices.shape
    assert h == _HIST and embeddings.shape[1] == _D
    pooler, nw, nchunk = _make_sc_pooler(b)
    idx3 = indices.astype(jnp.int32).reshape(nw, nchunk, _CHUNK_IDX)
    pooled = pooler(idx3, embeddings.astype(jnp.float32))
    return _normalize(pooled)
